# Initial kernel scaffold; baseline (speedup 1.0000x reference)
import jax
import jax.numpy as jnp
from jax.experimental import pallas as pl


def kernel(x, ln_gamma, ln_beta, rW1, rb1, rW2, rb2, eW1, eb1, eW2, eb2):
    raise NotImplementedError("write your pallas kernel here")


# speedup vs baseline: 1.2586x; 1.2586x over previous
import functools

import jax
import jax.numpy as jnp
from jax.experimental import pallas as pl
from jax.experimental.pallas import tpu as pltpu

N, D, H, E, K = 2048, 1024, 4096, 8, 2
H2 = H // 2
BT = 256
S = N * K + E * BT
NB = S // BT
TB = 256


def _router_kernel(x_ref, g_ref, b_ref, w1_ref, b1_ref, w2_ref, b2_ref,
                   xn_ref, pos_ref, wgt_ref, meta_ref):
    xb = x_ref[...]
    mu = jnp.mean(xb, axis=1, keepdims=True)
    xc = xb - mu
    var = jnp.mean(xc * xc, axis=1, keepdims=True)
    xn = xc / jnp.sqrt(var + 1e-5) * g_ref[...] + b_ref[...]
    xn_ref[...] = xn

    rh = jnp.dot(xn, w1_ref[...], precision=jax.lax.Precision.DEFAULT)
    rh = jnp.maximum(rh + b1_ref[...], 0.0)
    logits = jnp.dot(rh, w2_ref[...], precision=jax.lax.Precision.DEFAULT)
    logits = logits + b2_ref[...]

    m = jnp.max(logits, axis=1, keepdims=True)
    ex = jnp.exp(logits - m)
    probs = ex / jnp.sum(ex, axis=1, keepdims=True)

    lane = jax.lax.broadcasted_iota(jnp.int32, (N, E), 1)
    m0 = jnp.max(probs, axis=1, keepdims=True)
    i0 = jnp.min(jnp.where(probs == m0, lane, E), axis=1, keepdims=True)
    masked = jnp.where(lane == i0, -1.0, probs)
    m1 = jnp.max(masked, axis=1, keepdims=True)
    i1 = jnp.min(jnp.where(masked == m1, lane, E), axis=1, keepdims=True)
    ws = m0 + m1
    wgt_ref[...] = jnp.concatenate([m0 / ws, m1 / ws], axis=1)

    oh0 = (lane == i0).astype(jnp.float32)
    oh1 = (lane == i1).astype(jnp.float32)
    ohsum = oh0 + oh1
    s = ohsum
    sh = 1
    while sh < N:
        s = s + jnp.concatenate([jnp.zeros((sh, E), jnp.float32), s[:-sh]], axis=0)
        sh *= 2
    p_excl = s - ohsum
    counts = s[N - 1:N, :]
    pc = jnp.ceil(counts / BT) * BT
    r = jax.lax.broadcasted_iota(jnp.int32, (E, E), 0)
    c = jax.lax.broadcasted_iota(jnp.int32, (E, E), 1)
    upper = (r < c).astype(jnp.float32)
    offs = jnp.dot(pc, upper, precision=jax.lax.Precision.HIGHEST)
    base = p_excl + offs
    pos0 = jnp.sum(oh0 * base, axis=1, keepdims=True)
    pos1 = jnp.sum(oh1 * base, axis=1, keepdims=True)
    pos_ref[...] = jnp.concatenate([pos0, pos1], axis=1).astype(jnp.int32)
    sel_r = jax.lax.broadcasted_iota(jnp.int32, (E, 128), 0)
    sel_c = jax.lax.broadcasted_iota(jnp.int32, (E, 128), 1)
    sel = (sel_r == sel_c).astype(jnp.float32)
    meta0 = jnp.dot(offs + pc, sel, precision=jax.lax.Precision.HIGHEST)
    mrow = jax.lax.broadcasted_iota(jnp.int32, (8, 128), 0)
    meta_ref[...] = jnp.where(mrow == 0, jnp.broadcast_to(meta0, (8, 128)), 0.0)


def _ffn_kernel(be_ref, rid_ref, xn_ref, w1_ref, b1_ref, w2_ref, b2_ref,
                ys_ref, xs_scr):
    b = pl.program_id(0)
    be = be_ref[b]

    @pl.when(be < E)
    def _():
        def gather(i, carry):
            rid = rid_ref[b * BT + i]
            xs_scr[pl.ds(i, 1), :] = xn_ref[pl.ds(rid, 1), :]
            return carry
        jax.lax.fori_loop(0, BT, gather, 0)
        xs = xs_scr[...].astype(jnp.bfloat16)
        h = jnp.dot(xs, w1_ref[0], preferred_element_type=jnp.float32)
        h = jnp.maximum(h + b1_ref[0], 0.0).astype(jnp.bfloat16)
        y = jnp.dot(h, w2_ref[0], preferred_element_type=jnp.float32)
        ys_ref[...] = y + b2_ref[0]


def _combine_kernel(pos_ref, x_ref, wgt_ref, ys_ref, out_ref, g0_scr, g1_scr):
    t = pl.program_id(0)

    def gather(i, carry):
        p0 = pos_ref[(t * TB + i) * 2]
        p1 = pos_ref[(t * TB + i) * 2 + 1]
        g0_scr[pl.ds(i, 1), :] = ys_ref[pl.ds(p0, 1), :]
        g1_scr[pl.ds(i, 1), :] = ys_ref[pl.ds(p1, 1), :]
        return carry
    jax.lax.fori_loop(0, TB, gather, 0)
    out_ref[...] = (x_ref[...] + wgt_ref[:, 0:1] * g0_scr[...]
                    + wgt_ref[:, 1:2] * g1_scr[...])


@jax.jit
def kernel(x, ln_gamma, ln_beta, rW1, rb1, rW2, rb2, eW1, eb1, eW2, eb2):
    xn, pos, wgt, meta = pl.pallas_call(
        _router_kernel,
        out_shape=(
            jax.ShapeDtypeStruct((N, D), jnp.float32),
            jax.ShapeDtypeStruct((N, K), jnp.int32),
            jax.ShapeDtypeStruct((N, K), jnp.float32),
            jax.ShapeDtypeStruct((8, 128), jnp.float32),
        ),
    )(x, ln_gamma.reshape(1, D), ln_beta.reshape(1, D),
      rW1, rb1.reshape(1, H2), rW2, rb2.reshape(1, E))

    cum_pc = meta[0, :E].astype(jnp.int32)
    block_expert = jnp.searchsorted(
        cum_pc, jnp.arange(NB, dtype=jnp.int32) * BT, side='right'
    ).astype(jnp.int32)
    tok = jnp.arange(N, dtype=jnp.int32)
    row_ids = (jnp.zeros((S,), jnp.int32)
               .at[pos[:, 0]].set(tok)
               .at[pos[:, 1]].set(tok))

    w1b = eW1.astype(jnp.bfloat16)
    w2b = eW2.astype(jnp.bfloat16)

    ys = pl.pallas_call(
        _ffn_kernel,
        grid_spec=pltpu.PrefetchScalarGridSpec(
            num_scalar_prefetch=2,
            grid=(NB,),
            in_specs=[
                pl.BlockSpec((N, D), lambda b, be, rid: (0, 0)),
                pl.BlockSpec((1, D, H),
                             lambda b, be, rid: (jnp.minimum(be[b], E - 1), 0, 0)),
                pl.BlockSpec((1, 1, H),
                             lambda b, be, rid: (jnp.minimum(be[b], E - 1), 0, 0)),
                pl.BlockSpec((1, H, D),
                             lambda b, be, rid: (jnp.minimum(be[b], E - 1), 0, 0)),
                pl.BlockSpec((1, 1, D),
                             lambda b, be, rid: (jnp.minimum(be[b], E - 1), 0, 0)),
            ],
            out_specs=pl.BlockSpec((BT, D), lambda b, be, rid: (b, 0)),
            scratch_shapes=[pltpu.VMEM((BT, D), jnp.float32)],
        ),
        out_shape=jax.ShapeDtypeStruct((S, D), jnp.float32),
        compiler_params=pltpu.CompilerParams(
            dimension_semantics=("arbitrary",)),
    )(block_expert, row_ids, xn, w1b, eb1.reshape(E, 1, H), w2b,
      eb2.reshape(E, 1, D))

    out = pl.pallas_call(
        _combine_kernel,
        grid_spec=pltpu.PrefetchScalarGridSpec(
            num_scalar_prefetch=1,
            grid=(N // TB,),
            in_specs=[
                pl.BlockSpec((TB, D), lambda t, p: (t, 0)),
                pl.BlockSpec((TB, K), lambda t, p: (t, 0)),
                pl.BlockSpec((S, D), lambda t, p: (0, 0)),
            ],
            out_specs=pl.BlockSpec((TB, D), lambda t, p: (t, 0)),
            scratch_shapes=[pltpu.VMEM((TB, D), jnp.float32),
                            pltpu.VMEM((TB, D), jnp.float32)],
        ),
        out_shape=jax.ShapeDtypeStruct((N, D), jnp.float32),
        compiler_params=pltpu.CompilerParams(
            dimension_semantics=("arbitrary",)),
    )(pos.reshape(-1), x, wgt, ys)
    return out


# speedup vs baseline: 1.2828x; 1.0192x over previous
import functools

import jax
import jax.numpy as jnp
from jax.experimental import pallas as pl
from jax.experimental.pallas import tpu as pltpu

N, D, H, E, K = 2048, 1024, 4096, 8, 2
H2 = H // 2
BT = 256
S = N * K + E * BT
NB = S // BT
TB = 256


def _router_kernel(x_ref, g_ref, b_ref, w1_ref, b1_ref, w2_ref, b2_ref,
                   xn_ref, pos_ref, wgt_ref, meta_ref):
    xb = x_ref[...]
    mu = jnp.mean(xb, axis=1, keepdims=True)
    xc = xb - mu
    var = jnp.mean(xc * xc, axis=1, keepdims=True)
    xn = xc / jnp.sqrt(var + 1e-5) * g_ref[...] + b_ref[...]
    xn_ref[...] = xn

    rh = jnp.dot(xn, w1_ref[...], precision=jax.lax.Precision.DEFAULT)
    rh = jnp.maximum(rh + b1_ref[...], 0.0)
    logits = jnp.dot(rh, w2_ref[...], precision=jax.lax.Precision.DEFAULT)
    logits = logits + b2_ref[...]

    m = jnp.max(logits, axis=1, keepdims=True)
    ex = jnp.exp(logits - m)
    probs = ex / jnp.sum(ex, axis=1, keepdims=True)

    lane = jax.lax.broadcasted_iota(jnp.int32, (N, E), 1)
    m0 = jnp.max(probs, axis=1, keepdims=True)
    i0 = jnp.min(jnp.where(probs == m0, lane, E), axis=1, keepdims=True)
    masked = jnp.where(lane == i0, -1.0, probs)
    m1 = jnp.max(masked, axis=1, keepdims=True)
    i1 = jnp.min(jnp.where(masked == m1, lane, E), axis=1, keepdims=True)
    ws = m0 + m1
    wgt_ref[...] = jnp.concatenate([m0 / ws, m1 / ws], axis=1)

    oh0 = (lane == i0).astype(jnp.float32)
    oh1 = (lane == i1).astype(jnp.float32)
    ohsum = oh0 + oh1
    s = ohsum
    sh = 1
    while sh < N:
        s = s + jnp.concatenate([jnp.zeros((sh, E), jnp.float32), s[:-sh]], axis=0)
        sh *= 2
    p_excl = s - ohsum
    counts = s[N - 1:N, :]
    pc = jnp.ceil(counts / BT) * BT
    r = jax.lax.broadcasted_iota(jnp.int32, (E, E), 0)
    c = jax.lax.broadcasted_iota(jnp.int32, (E, E), 1)
    upper = (r < c).astype(jnp.float32)
    offs = jnp.dot(pc, upper, precision=jax.lax.Precision.HIGHEST)
    base = p_excl + offs
    pos0 = jnp.sum(oh0 * base, axis=1, keepdims=True)
    pos1 = jnp.sum(oh1 * base, axis=1, keepdims=True)
    pos_ref[...] = jnp.concatenate([pos0, pos1], axis=1).astype(jnp.int32)
    sel_r = jax.lax.broadcasted_iota(jnp.int32, (E, 128), 0)
    sel_c = jax.lax.broadcasted_iota(jnp.int32, (E, 128), 1)
    sel = (sel_r == sel_c).astype(jnp.float32)
    meta0 = jnp.dot(offs + pc, sel, precision=jax.lax.Precision.HIGHEST)
    mrow = jax.lax.broadcasted_iota(jnp.int32, (8, 128), 0)
    meta_ref[...] = jnp.where(mrow == 0, jnp.broadcast_to(meta0, (8, 128)), 0.0)


def _ffn_kernel(be_ref, rid_ref, xn_ref, w1_ref, b1_ref, w2_ref, b2_ref,
                ys_ref, xs_scr):
    hb = pl.program_id(0)
    b = pl.program_id(1)
    be = be_ref[b]

    @pl.when(be < E)
    def _():
        def gather(i, carry):
            rid = rid_ref[b * BT + i]
            xs_scr[pl.ds(i, 1), :] = xn_ref[pl.ds(rid, 1), :]
            return carry
        jax.lax.fori_loop(0, BT, gather, 0)
        xs = xs_scr[...]
        h = jnp.dot(xs, w1_ref[0], precision=jax.lax.Precision.DEFAULT)
        h = jnp.maximum(h + b1_ref[0], 0.0)
        y = jnp.dot(h, w2_ref[0], precision=jax.lax.Precision.DEFAULT)
        ys_ref[...] = y + b2_ref[0] * (1.0 - hb.astype(jnp.float32))


def _combine_kernel(pos_ref, x_ref, wgt_ref, ys_ref, out_ref, g0_scr, g1_scr):
    t = pl.program_id(0)

    def gather(i, carry):
        p0 = pos_ref[(t * TB + i) * 2]
        p1 = pos_ref[(t * TB + i) * 2 + 1]
        g0_scr[pl.ds(i, 1), :] = ys_ref[pl.ds(p0, 1), :] + ys_ref[pl.ds(S + p0, 1), :]
        g1_scr[pl.ds(i, 1), :] = ys_ref[pl.ds(p1, 1), :] + ys_ref[pl.ds(S + p1, 1), :]
        return carry
    jax.lax.fori_loop(0, TB, gather, 0)
    out_ref[...] = (x_ref[...] + wgt_ref[:, 0:1] * g0_scr[...]
                    + wgt_ref[:, 1:2] * g1_scr[...])


@jax.jit
def kernel(x, ln_gamma, ln_beta, rW1, rb1, rW2, rb2, eW1, eb1, eW2, eb2):
    xn, pos, wgt, meta = pl.pallas_call(
        _router_kernel,
        out_shape=(
            jax.ShapeDtypeStruct((N, D), jnp.float32),
            jax.ShapeDtypeStruct((N, K), jnp.int32),
            jax.ShapeDtypeStruct((N, K), jnp.float32),
            jax.ShapeDtypeStruct((8, 128), jnp.float32),
        ),
    )(x, ln_gamma.reshape(1, D), ln_beta.reshape(1, D),
      rW1, rb1.reshape(1, H2), rW2, rb2.reshape(1, E))

    cum_pc = meta[0, :E].astype(jnp.int32)
    block_expert = jnp.searchsorted(
        cum_pc, jnp.arange(NB, dtype=jnp.int32) * BT, side='right'
    ).astype(jnp.int32)
    tok = jnp.arange(N, dtype=jnp.int32)
    row_ids = (jnp.zeros((S,), jnp.int32)
               .at[pos[:, 0]].set(tok)
               .at[pos[:, 1]].set(tok))

    ys = pl.pallas_call(
        _ffn_kernel,
        grid_spec=pltpu.PrefetchScalarGridSpec(
            num_scalar_prefetch=2,
            grid=(2, NB),
            in_specs=[
                pl.BlockSpec((N, D), lambda hb, b, be, rid: (0, 0)),
                pl.BlockSpec((1, D, H // 2),
                             lambda hb, b, be, rid: (jnp.minimum(be[b], E - 1), 0, hb)),
                pl.BlockSpec((1, 1, H // 2),
                             lambda hb, b, be, rid: (jnp.minimum(be[b], E - 1), 0, hb)),
                pl.BlockSpec((1, H // 2, D),
                             lambda hb, b, be, rid: (jnp.minimum(be[b], E - 1), hb, 0)),
                pl.BlockSpec((1, 1, D),
                             lambda hb, b, be, rid: (jnp.minimum(be[b], E - 1), 0, 0)),
            ],
            out_specs=pl.BlockSpec((BT, D), lambda hb, b, be, rid: (hb * NB + b, 0)),
            scratch_shapes=[pltpu.VMEM((BT, D), jnp.float32)],
        ),
        out_shape=jax.ShapeDtypeStruct((2 * S, D), jnp.float32),
        compiler_params=pltpu.CompilerParams(
            dimension_semantics=("arbitrary", "arbitrary"),
            vmem_limit_bytes=60 * 1024 * 1024),
    )(block_expert, row_ids, xn, eW1, eb1.reshape(E, 1, H), eW2,
      eb2.reshape(E, 1, D))

    out = pl.pallas_call(
        _combine_kernel,
        grid_spec=pltpu.PrefetchScalarGridSpec(
            num_scalar_prefetch=1,
            grid=(N // TB,),
            in_specs=[
                pl.BlockSpec((TB, D), lambda t, p: (t, 0)),
                pl.BlockSpec((TB, K), lambda t, p: (t, 0)),
                pl.BlockSpec((2 * S, D), lambda t, p: (0, 0)),
            ],
            out_specs=pl.BlockSpec((TB, D), lambda t, p: (t, 0)),
            scratch_shapes=[pltpu.VMEM((TB, D), jnp.float32),
                            pltpu.VMEM((TB, D), jnp.float32)],
        ),
        out_shape=jax.ShapeDtypeStruct((N, D), jnp.float32),
        compiler_params=pltpu.CompilerParams(
            dimension_semantics=("arbitrary",),
            vmem_limit_bytes=62 * 1024 * 1024),
    )(pos.reshape(-1), x, wgt, ys)
    return out


# speedup vs baseline: 1.2930x; 1.0080x over previous
import functools

import jax
from jax import lax
import jax.numpy as jnp
from jax.experimental import pallas as pl
from jax.experimental.pallas import tpu as pltpu
from jax.experimental.pallas import tpu_sc as plsc

N, D, H, E, K = 2048, 1024, 4096, 8, 2
H2 = H // 2
BT = 256
S = N * K + E * BT
NB = S // BT
TB = 256
NW = 32
CH = N // NW
CH2 = CH // 2


def _router_kernel(x_ref, g_ref, b_ref, w1_ref, b1_ref, w2_ref, b2_ref,
                   xn_ref, pos_ref, wgt_ref, meta_ref):
    xb = x_ref[...]
    mu = jnp.mean(xb, axis=1, keepdims=True)
    xc = xb - mu
    var = jnp.mean(xc * xc, axis=1, keepdims=True)
    xn = xc / jnp.sqrt(var + 1e-5) * g_ref[...] + b_ref[...]
    xn_ref[...] = xn

    rh = jnp.dot(xn, w1_ref[...], precision=jax.lax.Precision.DEFAULT)
    rh = jnp.maximum(rh + b1_ref[...], 0.0)
    logits = jnp.dot(rh, w2_ref[...], precision=jax.lax.Precision.DEFAULT)
    logits = logits + b2_ref[...]

    m = jnp.max(logits, axis=1, keepdims=True)
    ex = jnp.exp(logits - m)
    probs = ex / jnp.sum(ex, axis=1, keepdims=True)

    lane = jax.lax.broadcasted_iota(jnp.int32, (N, E), 1)
    m0 = jnp.max(probs, axis=1, keepdims=True)
    i0 = jnp.min(jnp.where(probs == m0, lane, E), axis=1, keepdims=True)
    masked = jnp.where(lane == i0, -1.0, probs)
    m1 = jnp.max(masked, axis=1, keepdims=True)
    i1 = jnp.min(jnp.where(masked == m1, lane, E), axis=1, keepdims=True)
    ws = m0 + m1
    wgt_ref[...] = jnp.concatenate([m0 / ws, m1 / ws], axis=1)

    oh0 = (lane == i0).astype(jnp.float32)
    oh1 = (lane == i1).astype(jnp.float32)
    ohsum = oh0 + oh1
    s = ohsum
    sh = 1
    while sh < N:
        s = s + jnp.concatenate([jnp.zeros((sh, E), jnp.float32), s[:-sh]], axis=0)
        sh *= 2
    p_excl = s - ohsum
    counts = s[N - 1:N, :]
    pc = jnp.ceil(counts / BT) * BT
    r = jax.lax.broadcasted_iota(jnp.int32, (E, E), 0)
    c = jax.lax.broadcasted_iota(jnp.int32, (E, E), 1)
    upper = (r < c).astype(jnp.float32)
    offs = jnp.dot(pc, upper, precision=jax.lax.Precision.HIGHEST)
    base = p_excl + offs
    pos0 = jnp.sum(oh0 * base, axis=1, keepdims=True)
    pos1 = jnp.sum(oh1 * base, axis=1, keepdims=True)
    pos_ref[...] = jnp.concatenate([pos0, pos1], axis=1).astype(jnp.int32)
    sel_r = jax.lax.broadcasted_iota(jnp.int32, (E, 128), 0)
    sel_c = jax.lax.broadcasted_iota(jnp.int32, (E, 128), 1)
    sel = (sel_r == sel_c).astype(jnp.float32)
    meta0 = jnp.dot(offs + pc, sel, precision=jax.lax.Precision.HIGHEST)
    mrow = jax.lax.broadcasted_iota(jnp.int32, (8, 128), 0)
    meta_ref[...] = jnp.where(mrow == 0, jnp.broadcast_to(meta0, (8, 128)), 0.0)


def _dispatch_kernel(xn_hbm, p0_hbm, p1_hbm, xs_hbm, idx_v, rows_v, sem):
    wid = lax.axis_index("s") * 2 + lax.axis_index("c")
    base = wid * CH
    pltpu.sync_copy(xn_hbm.at[pl.ds(base, CH)], rows_v)
    pltpu.sync_copy(p0_hbm.at[pl.ds(base, CH)], idx_v)
    pltpu.async_copy(rows_v, xs_hbm.at[idx_v], sem).wait()
    pltpu.sync_copy(p1_hbm.at[pl.ds(base, CH)], idx_v)
    pltpu.async_copy(rows_v, xs_hbm.at[idx_v], sem).wait()


def _ffn_kernel(be_ref, xs_ref, w1_ref, b1_ref, w2_ref, b2_ref, ys_ref):
    hb = pl.program_id(0)
    b = pl.program_id(1)
    be = be_ref[b]

    @pl.when(be < E)
    def _():
        xs = xs_ref[...]
        h = jnp.dot(xs, w1_ref[0], precision=jax.lax.Precision.DEFAULT)
        h = jnp.maximum(h + b1_ref[0], 0.0)
        y = jnp.dot(h, w2_ref[0], precision=jax.lax.Precision.DEFAULT)
        ys_ref[...] = y + b2_ref[0] * (1.0 - hb.astype(jnp.float32))


def _sc_combine_kernel(x_hbm, w0_hbm, w1_hbm, p0a_hbm, p0b_hbm, p1a_hbm,
                       p1b_hbm, ys_hbm, out_hbm,
                       idx_v, w0_v, w1_v, acc_v, g_v, sem):
    wid = lax.axis_index("s") * 2 + lax.axis_index("c")
    for half in range(2):
        base = wid * CH + half * CH2
        pltpu.sync_copy(x_hbm.at[pl.ds(base, CH2)], acc_v)
        pltpu.sync_copy(w0_hbm.at[pl.ds(base, CH2)], w0_v)
        pltpu.sync_copy(w1_hbm.at[pl.ds(base, CH2)], w1_v)
        for p_hbm, w_v in ((p0a_hbm, w0_v), (p0b_hbm, w0_v),
                           (p1a_hbm, w1_v), (p1b_hbm, w1_v)):
            pltpu.sync_copy(p_hbm.at[pl.ds(base, CH2)], idx_v)
            pltpu.async_copy(ys_hbm.at[idx_v], g_v, sem).wait()

            def tok_body(i, carry):
                wspl = plsc.load_gather(w_v, [jnp.zeros((16,), jnp.int32) + i])

                def chunk_body(c, inner):
                    sl = pl.ds(c * 16, 16)
                    acc_v[i, sl] = acc_v[i, sl] + wspl * g_v[i, sl]
                    return inner
                return jax.lax.fori_loop(0, D // 16, chunk_body, carry)
            jax.lax.fori_loop(0, CH2, tok_body, 0)
        pltpu.sync_copy(acc_v, out_hbm.at[pl.ds(base, CH2)])


@jax.jit
def kernel(x, ln_gamma, ln_beta, rW1, rb1, rW2, rb2, eW1, eb1, eW2, eb2):
    xn, pos, wgt, meta = pl.pallas_call(
        _router_kernel,
        out_shape=(
            jax.ShapeDtypeStruct((N, D), jnp.float32),
            jax.ShapeDtypeStruct((N, K), jnp.int32),
            jax.ShapeDtypeStruct((N, K), jnp.float32),
            jax.ShapeDtypeStruct((8, 128), jnp.float32),
        ),
    )(x, ln_gamma.reshape(1, D), ln_beta.reshape(1, D),
      rW1, rb1.reshape(1, H2), rW2, rb2.reshape(1, E))

    cum_pc = meta[0, :E].astype(jnp.int32)
    block_expert = jnp.searchsorted(
        cum_pc, jnp.arange(NB, dtype=jnp.int32) * BT, side='right'
    ).astype(jnp.int32)

    mesh = plsc.VectorSubcoreMesh(core_axis_name="c", subcore_axis_name="s")
    xs = pl.kernel(
        _dispatch_kernel,
        out_type=jax.ShapeDtypeStruct((S, D), jnp.float32),
        mesh=mesh,
        scratch_types=[pltpu.VMEM((CH,), jnp.int32),
                       pltpu.VMEM((CH, D), jnp.float32),
                       pltpu.SemaphoreType.DMA],
    )(xn, pos[:, 0], pos[:, 1])

    ys = pl.pallas_call(
        _ffn_kernel,
        grid_spec=pltpu.PrefetchScalarGridSpec(
            num_scalar_prefetch=1,
            grid=(2, NB),
            in_specs=[
                pl.BlockSpec((BT, D), lambda hb, b, be: (b, 0)),
                pl.BlockSpec((1, D, H // 2),
                             lambda hb, b, be: (jnp.minimum(be[b], E - 1), 0, hb)),
                pl.BlockSpec((1, 1, H // 2),
                             lambda hb, b, be: (jnp.minimum(be[b], E - 1), 0, hb)),
                pl.BlockSpec((1, H // 2, D),
                             lambda hb, b, be: (jnp.minimum(be[b], E - 1), hb, 0)),
                pl.BlockSpec((1, 1, D),
                             lambda hb, b, be: (jnp.minimum(be[b], E - 1), 0, 0)),
            ],
            out_specs=pl.BlockSpec((BT, D), lambda hb, b, be: (hb * NB + b, 0)),
        ),
        out_shape=jax.ShapeDtypeStruct((2 * S, D), jnp.float32),
        compiler_params=pltpu.CompilerParams(
            dimension_semantics=("arbitrary", "arbitrary"),
            vmem_limit_bytes=60 * 1024 * 1024),
    )(block_expert, xs, eW1, eb1.reshape(E, 1, H), eW2, eb2.reshape(E, 1, D))

    p0 = pos[:, 0]
    p1 = pos[:, 1]
    out = pl.kernel(
        _sc_combine_kernel,
        out_type=jax.ShapeDtypeStruct((N, D), jnp.float32),
        mesh=mesh,
        scratch_types=[pltpu.VMEM((CH2,), jnp.int32),
                       pltpu.VMEM((CH2,), jnp.float32),
                       pltpu.VMEM((CH2,), jnp.float32),
                       pltpu.VMEM((CH2, D), jnp.float32),
                       pltpu.VMEM((CH2, D), jnp.float32),
                       pltpu.SemaphoreType.DMA],
        compiler_params=pltpu.CompilerParams(needs_layout_passes=False),
    )(x, wgt[:, 0], wgt[:, 1], p0, p0 + S, p1, p1 + S, ys)
    return out


# speedup vs baseline: 1.5436x; 1.1938x over previous
import functools

import jax
from jax import lax
import jax.numpy as jnp
from jax.experimental import pallas as pl
from jax.experimental.pallas import tpu as pltpu
from jax.experimental.pallas import tpu_sc as plsc

N, D, H, E, K = 2048, 1024, 4096, 8, 2
H2 = H // 2
BT = 256
S = N * K + E * BT
NB = S // BT
TB = 256
NW = 32
CH = N // NW
CH2 = CH // 2


def _router_kernel(x_ref, g_ref, b_ref, w1_ref, b1_ref, w2_ref, b2_ref,
                   xn_ref, pos_ref, wgt_ref, meta_ref):
    xb = x_ref[...]
    mu = jnp.mean(xb, axis=1, keepdims=True)
    xc = xb - mu
    var = jnp.mean(xc * xc, axis=1, keepdims=True)
    xn = xc / jnp.sqrt(var + 1e-5) * g_ref[...] + b_ref[...]
    xn_ref[...] = xn

    rh = jnp.dot(xn, w1_ref[...], precision=jax.lax.Precision.DEFAULT)
    rh = jnp.maximum(rh + b1_ref[...], 0.0)
    logits = jnp.dot(rh, w2_ref[...], precision=jax.lax.Precision.DEFAULT)
    logits = logits + b2_ref[...]

    m = jnp.max(logits, axis=1, keepdims=True)
    ex = jnp.exp(logits - m)
    probs = ex / jnp.sum(ex, axis=1, keepdims=True)

    lane = jax.lax.broadcasted_iota(jnp.int32, (N, E), 1)
    m0 = jnp.max(probs, axis=1, keepdims=True)
    i0 = jnp.min(jnp.where(probs == m0, lane, E), axis=1, keepdims=True)
    masked = jnp.where(lane == i0, -1.0, probs)
    m1 = jnp.max(masked, axis=1, keepdims=True)
    i1 = jnp.min(jnp.where(masked == m1, lane, E), axis=1, keepdims=True)
    ws = m0 + m1
    wgt_ref[...] = jnp.concatenate([m0 / ws, m1 / ws], axis=1)

    oh0 = (lane == i0).astype(jnp.float32)
    oh1 = (lane == i1).astype(jnp.float32)
    ohsum = oh0 + oh1
    s = ohsum
    sh = 1
    while sh < N:
        s = s + jnp.concatenate([jnp.zeros((sh, E), jnp.float32), s[:-sh]], axis=0)
        sh *= 2
    p_excl = s - ohsum
    counts = s[N - 1:N, :]
    pc = jnp.ceil(counts / BT) * BT
    r = jax.lax.broadcasted_iota(jnp.int32, (E, E), 0)
    c = jax.lax.broadcasted_iota(jnp.int32, (E, E), 1)
    upper = (r < c).astype(jnp.float32)
    offs = jnp.dot(pc, upper, precision=jax.lax.Precision.HIGHEST)
    base = p_excl + offs
    pos0 = jnp.sum(oh0 * base, axis=1, keepdims=True)
    pos1 = jnp.sum(oh1 * base, axis=1, keepdims=True)
    pos_ref[...] = jnp.concatenate([pos0, pos1], axis=1).astype(jnp.int32)
    sel_r = jax.lax.broadcasted_iota(jnp.int32, (E, 128), 0)
    sel_c = jax.lax.broadcasted_iota(jnp.int32, (E, 128), 1)
    sel = (sel_r == sel_c).astype(jnp.float32)
    meta0 = jnp.dot(offs + pc, sel, precision=jax.lax.Precision.HIGHEST)
    mrow = jax.lax.broadcasted_iota(jnp.int32, (8, 128), 0)
    meta_ref[...] = jnp.where(mrow == 0, jnp.broadcast_to(meta0, (8, 128)), 0.0)


def _col16(mat_v, k, col):
    rows = jax.lax.iota(jnp.int32, 16) + k * 16
    cols = jnp.zeros((16,), jnp.int32) + col
    return plsc.load_gather(mat_v, [rows, cols])


def _dispatch_kernel(xn_hbm, pos_hbm, xs_hbm, pos_v, idx_v, rows_v, sem):
    wid = lax.axis_index("s") * 2 + lax.axis_index("c")
    base = wid * CH
    pltpu.sync_copy(xn_hbm.at[pl.ds(base, CH)], rows_v)
    pltpu.sync_copy(pos_hbm.at[pl.ds(base, CH)], pos_v)
    for col in range(K):
        for k in range(CH // 16):
            idx_v[pl.ds(k * 16, 16)] = _col16(pos_v, k, col)
        pltpu.async_copy(rows_v, xs_hbm.at[idx_v], sem).wait()


def _ffn_kernel(be_ref, xs_ref, w1_ref, b1_ref, w2_ref, b2_ref, ys_ref):
    hb = pl.program_id(0)
    b = pl.program_id(1)
    be = be_ref[b]

    @pl.when(be < E)
    def _():
        xs = xs_ref[...]
        h = jnp.dot(xs, w1_ref[0], precision=jax.lax.Precision.DEFAULT)
        h = jnp.maximum(h + b1_ref[0], 0.0)
        y = jnp.dot(h, w2_ref[0], precision=jax.lax.Precision.DEFAULT)
        ys_ref[...] = y + b2_ref[0] * (1.0 - hb.astype(jnp.float32))


def _sc_combine_kernel(x_hbm, wgt_hbm, pos_hbm, ys_hbm, out_hbm,
                       pos_v, wgt_v, idx_v, acc_v, g_v, sem):
    wid = lax.axis_index("s") * 2 + lax.axis_index("c")
    for half in range(2):
        base = wid * CH + half * CH2
        pltpu.sync_copy(x_hbm.at[pl.ds(base, CH2)], acc_v)
        pltpu.sync_copy(wgt_hbm.at[pl.ds(base, CH2)], wgt_v)
        pltpu.sync_copy(pos_hbm.at[pl.ds(base, CH2)], pos_v)
        for col in range(K):
            for second in range(2):
                for k in range(CH2 // 16):
                    c16 = _col16(pos_v, k, col) + second * S
                    idx_v[pl.ds(k * 16, 16)] = c16
                pltpu.async_copy(ys_hbm.at[idx_v], g_v, sem).wait()

                def tok_body(i, carry):
                    wspl = plsc.load_gather(
                        wgt_v, [jnp.zeros((16,), jnp.int32) + i,
                                jnp.zeros((16,), jnp.int32) + col])
                    for c in range(D // 16):
                        sl = pl.ds(c * 16, 16)
                        acc_v[i, sl] = acc_v[i, sl] + wspl * g_v[i, sl]
                    return carry
                jax.lax.fori_loop(0, CH2, tok_body, 0)
        pltpu.sync_copy(acc_v, out_hbm.at[pl.ds(base, CH2)])


@jax.jit
def kernel(x, ln_gamma, ln_beta, rW1, rb1, rW2, rb2, eW1, eb1, eW2, eb2):
    xn, pos, wgt, meta = pl.pallas_call(
        _router_kernel,
        out_shape=(
            jax.ShapeDtypeStruct((N, D), jnp.float32),
            jax.ShapeDtypeStruct((N, K), jnp.int32),
            jax.ShapeDtypeStruct((N, K), jnp.float32),
            jax.ShapeDtypeStruct((8, 128), jnp.float32),
        ),
    )(x, ln_gamma.reshape(1, D), ln_beta.reshape(1, D),
      rW1, rb1.reshape(1, H2), rW2, rb2.reshape(1, E))

    cum_pc = meta[0, :E].astype(jnp.int32)
    block_expert = jnp.searchsorted(
        cum_pc, jnp.arange(NB, dtype=jnp.int32) * BT, side='right'
    ).astype(jnp.int32)

    mesh = plsc.VectorSubcoreMesh(core_axis_name="c", subcore_axis_name="s")
    xs = pl.kernel(
        _dispatch_kernel,
        out_type=jax.ShapeDtypeStruct((S, D), jnp.float32),
        mesh=mesh,
        scratch_types=[pltpu.VMEM((CH, K), jnp.int32),
                       pltpu.VMEM((CH,), jnp.int32),
                       pltpu.VMEM((CH, D), jnp.float32),
                       pltpu.SemaphoreType.DMA],
        compiler_params=pltpu.CompilerParams(needs_layout_passes=False),
    )(xn, pos)

    ys = pl.pallas_call(
        _ffn_kernel,
        grid_spec=pltpu.PrefetchScalarGridSpec(
            num_scalar_prefetch=1,
            grid=(2, NB),
            in_specs=[
                pl.BlockSpec((BT, D), lambda hb, b, be: (b, 0)),
                pl.BlockSpec((1, D, H // 2),
                             lambda hb, b, be: (jnp.minimum(be[b], E - 1), 0, hb)),
                pl.BlockSpec((1, 1, H // 2),
                             lambda hb, b, be: (jnp.minimum(be[b], E - 1), 0, hb)),
                pl.BlockSpec((1, H // 2, D),
                             lambda hb, b, be: (jnp.minimum(be[b], E - 1), hb, 0)),
                pl.BlockSpec((1, 1, D),
                             lambda hb, b, be: (jnp.minimum(be[b], E - 1), 0, 0)),
            ],
            out_specs=pl.BlockSpec((BT, D), lambda hb, b, be: (hb * NB + b, 0)),
        ),
        out_shape=jax.ShapeDtypeStruct((2 * S, D), jnp.float32),
        compiler_params=pltpu.CompilerParams(
            dimension_semantics=("arbitrary", "arbitrary"),
            vmem_limit_bytes=60 * 1024 * 1024),
    )(block_expert, xs, eW1, eb1.reshape(E, 1, H), eW2, eb2.reshape(E, 1, D))

    out = pl.kernel(
        _sc_combine_kernel,
        out_type=jax.ShapeDtypeStruct((N, D), jnp.float32),
        mesh=mesh,
        scratch_types=[pltpu.VMEM((CH2, K), jnp.int32),
                       pltpu.VMEM((CH2, K), jnp.float32),
                       pltpu.VMEM((CH2,), jnp.int32),
                       pltpu.VMEM((CH2, D), jnp.float32),
                       pltpu.VMEM((CH2, D), jnp.float32),
                       pltpu.SemaphoreType.DMA],
        compiler_params=pltpu.CompilerParams(needs_layout_passes=False),
    )(x, wgt, pos, ys)
    return out
